# Initial kernel scaffold; baseline (speedup 1.0000x reference)
import functools

import jax
import jax.numpy as jnp
from jax import lax
from jax.experimental import pallas as pl
from jax.experimental.pallas import tpu as pltpu
from jax.experimental.pallas import tpu_sc as plsc

SPLIT_RATIO = 0.5
NUM_ITERS = 3

_INFO = plsc.get_sparse_core_info()
NC = _INFO.num_cores
NS = _INFO.num_subcores
NW = NC * NS
CHUNK = 128

ROW_BLK = 1280


def _round_up(v, m):
    return (v + m - 1) // m * m


@functools.partial(jax.jit, static_argnames=("npad", "width", "chunk"))
def _sc_scatter_add(idx_pad, rows_pad, zeros, *, npad, width, chunk):
    epad = idx_pad.shape[0]
    epw = epad // NW
    nchunk = epw // chunk
    zrows = npad // NS

    mesh = plsc.VectorSubcoreMesh(core_axis_name="c", subcore_axis_name="s")

    @functools.partial(
        pl.kernel,
        out_type=jax.ShapeDtypeStruct((NC, npad, width), jnp.float32),
        mesh=mesh,
        scratch_types=[
            pltpu.VMEM((chunk,), jnp.int32),
            pltpu.VMEM((chunk, width), jnp.float32),
            pltpu.VMEM_SHARED((npad, width), jnp.float32),
        ],
    )
    def body(idx_hbm, rows_hbm, zeros_hbm, out_hbm, didx, rows_v, agg_sh):
        cid = lax.axis_index("c")
        sid = lax.axis_index("s")
        wid = sid * NC + cid
        pltpu.sync_copy(zeros_hbm.at[pl.ds(sid * zrows, zrows)],
                        agg_sh.at[pl.ds(sid * zrows, zrows)])
        plsc.subcore_barrier()

        def step(k, carry):
            base = wid * epw + k * chunk
            pltpu.sync_copy(idx_hbm.at[pl.ds(base, chunk)], didx)
            pltpu.sync_copy(rows_hbm.at[pl.ds(base, chunk)], rows_v)
            pltpu.sync_copy(rows_v, agg_sh.at[didx], add=True)
            return carry

        lax.fori_loop(0, nchunk, step, 0)
        plsc.subcore_barrier()
        pltpu.sync_copy(agg_sh.at[pl.ds(sid * zrows, zrows)],
                        out_hbm.at[cid].at[pl.ds(sid * zrows, zrows)])

    return body(idx_pad, rows_pad, zeros)


@functools.partial(jax.jit, static_argnames=("npad", "width", "chunk"))
def _sc_gather_scatter_add(h_aug, src_pad, dst_pad, zeros, *, npad, width, chunk):
    epad = src_pad.shape[0]
    epw = epad // NW
    nchunk = epw // chunk
    zrows = npad // NS

    mesh = plsc.VectorSubcoreMesh(core_axis_name="c", subcore_axis_name="s")

    @functools.partial(
        pl.kernel,
        out_type=jax.ShapeDtypeStruct((NC, npad, width), jnp.float32),
        mesh=mesh,
        scratch_types=[
            pltpu.VMEM((chunk,), jnp.int32),
            pltpu.VMEM((chunk,), jnp.int32),
            pltpu.VMEM((chunk, width), jnp.float32),
            pltpu.VMEM_SHARED((npad, width), jnp.float32),
            pltpu.SemaphoreType.DMA,
        ],
    )
    def body(h_hbm, src_hbm, dst_hbm, zeros_hbm, out_hbm,
             sidx, didx, rows_v, agg_sh, sem):
        cid = lax.axis_index("c")
        sid = lax.axis_index("s")
        wid = sid * NC + cid
        pltpu.sync_copy(zeros_hbm.at[pl.ds(sid * zrows, zrows)],
                        agg_sh.at[pl.ds(sid * zrows, zrows)])
        plsc.subcore_barrier()

        def step(k, carry):
            base = wid * epw + k * chunk
            pltpu.sync_copy(src_hbm.at[pl.ds(base, chunk)], sidx)
            pltpu.sync_copy(dst_hbm.at[pl.ds(base, chunk)], didx)
            pltpu.async_copy(h_hbm.at[sidx], rows_v, sem).wait()
            pltpu.sync_copy(rows_v, agg_sh.at[didx], add=True)
            return carry

        lax.fori_loop(0, nchunk, step, 0)
        plsc.subcore_barrier()
        pltpu.sync_copy(agg_sh.at[pl.ds(sid * zrows, zrows)],
                        out_hbm.at[cid].at[pl.ds(sid * zrows, zrows)])

    return body(h_aug, src_pad, dst_pad, zeros)


def _tc_h1(x_pad, y2, flagsp, W, b2, *, d, c, width):
    npad = x_pad.shape[0]
    grid = npad // ROW_BLK

    def body(x_ref, y_ref, fl_ref, w_ref, b_ref, haug_ref, l_ref, h0_ref):
        xb = x_ref[...]
        w1 = w_ref[:d, :]
        w2 = w_ref[d:, :]
        h0 = jnp.dot(xb, w1, preferred_element_type=jnp.float32) + b_ref[...]
        labeled = (fl_ref[0, :, 0:1] + fl_ref[1, :, 0:1]) > 0.0
        iota = lax.broadcasted_iota(jnp.int32, (ROW_BLK, c), 1)
        onehot = jnp.where(labeled & (y_ref[...] == iota), 1.0, 0.0)
        h = h0 + jnp.dot(onehot, w2, preferred_element_type=jnp.float32)
        haug_ref[...] = jnp.concatenate(
            [h, jnp.ones((ROW_BLK, 1), jnp.float32),
             jnp.zeros((ROW_BLK, width - c - 1), jnp.float32)], axis=1)
        l_ref[...] = onehot
        h0_ref[...] = h0

    return pl.pallas_call(
        body,
        grid=(grid,),
        in_specs=[
            pl.BlockSpec((ROW_BLK, d), lambda i: (i, 0)),
            pl.BlockSpec((ROW_BLK, 1), lambda i: (i, 0)),
            pl.BlockSpec((2, ROW_BLK, 8), lambda i: (0, i, 0)),
            pl.BlockSpec(W.shape, lambda i: (0, 0)),
            pl.BlockSpec(b2.shape, lambda i: (0, 0)),
        ],
        out_specs=[
            pl.BlockSpec((ROW_BLK, width), lambda i: (i, 0)),
            pl.BlockSpec((ROW_BLK, c), lambda i: (i, 0)),
            pl.BlockSpec((ROW_BLK, c), lambda i: (i, 0)),
        ],
        out_shape=[
            jax.ShapeDtypeStruct((npad, width), jnp.float32),
            jax.ShapeDtypeStruct((npad, c), jnp.float32),
            jax.ShapeDtypeStruct((npad, c), jnp.float32),
        ],
    )(x_pad, y2, flagsp, W, b2)


def _tc_iter(aggp, h0_pad, l_pad, flagsp, W, *, d, c, width):
    npad = h0_pad.shape[0]
    grid = npad // ROW_BLK

    def body(agg_ref, h0_ref, l_ref, fl_ref, w_ref, haug_ref, l_out_ref):
        agg = agg_ref[0] + agg_ref[1]
        deg = jnp.maximum(agg[:, c:c + 1], 1.0)
        out = agg[:, :c] / deg
        m = jnp.max(out, axis=1, keepdims=True)
        ex = jnp.exp(out - m)
        p = ex / jnp.sum(ex, axis=1, keepdims=True)
        upd = (fl_ref[0, :, 1:2] + fl_ref[1, :, 1:2]) > 0.0
        l_new = jnp.where(upd, p, l_ref[...])
        w2 = w_ref[d:, :]
        h = h0_ref[...] + jnp.dot(l_new, w2, preferred_element_type=jnp.float32)
        haug_ref[...] = jnp.concatenate(
            [h, jnp.ones((ROW_BLK, 1), jnp.float32),
             jnp.zeros((ROW_BLK, width - c - 1), jnp.float32)], axis=1)
        l_out_ref[...] = l_new

    return pl.pallas_call(
        body,
        grid=(grid,),
        in_specs=[
            pl.BlockSpec((2, ROW_BLK, width), lambda i: (0, i, 0)),
            pl.BlockSpec((ROW_BLK, c), lambda i: (i, 0)),
            pl.BlockSpec((ROW_BLK, c), lambda i: (i, 0)),
            pl.BlockSpec((2, ROW_BLK, 8), lambda i: (0, i, 0)),
            pl.BlockSpec(W.shape, lambda i: (0, 0)),
        ],
        out_specs=[
            pl.BlockSpec((ROW_BLK, width), lambda i: (i, 0)),
            pl.BlockSpec((ROW_BLK, c), lambda i: (i, 0)),
        ],
        out_shape=[
            jax.ShapeDtypeStruct((npad, width), jnp.float32),
            jax.ShapeDtypeStruct((npad, c), jnp.float32),
        ],
    )(aggp, h0_pad, l_pad, flagsp, W)


def _tc_final(aggp, *, c, width):
    npad = aggp.shape[1]
    grid = npad // ROW_BLK

    def body(agg_ref, out_ref):
        agg = agg_ref[0] + agg_ref[1]
        deg = jnp.maximum(agg[:, c:c + 1], 1.0)
        out_ref[...] = agg[:, :c] / deg

    return pl.pallas_call(
        body,
        grid=(grid,),
        in_specs=[pl.BlockSpec((2, ROW_BLK, width), lambda i: (0, i, 0))],
        out_specs=pl.BlockSpec((ROW_BLK, c), lambda i: (i, 0)),
        out_shape=jax.ShapeDtypeStruct((npad, c), jnp.float32),
    )(aggp)


def kernel(x, edge_index, y, train_idx, W, b):
    n, d = x.shape
    c = W.shape[0] - d
    e = edge_index.shape[1]
    nt = train_idx.shape[0]
    width = _round_up(c + 1, 16)
    npad = _round_up(n + 1, ROW_BLK)
    epad = _round_up(e, NW * CHUNK)
    ntpad = _round_up(nt, NW * 16)

    f32 = jnp.float32
    mask = jax.random.uniform(jax.random.key(1), (nt,)) < SPLIT_RATIO
    mask_f = mask.astype(f32)

    tidx_pad = jnp.full((ntpad,), n, jnp.int32).at[:nt].set(train_idx)
    m8 = (jnp.zeros((ntpad, 8), f32)
          .at[:nt, 0].set(mask_f)
          .at[:nt, 1].set(1.0 - mask_f))
    src_pad = jnp.full((epad,), n, jnp.int32).at[:e].set(edge_index[0])
    dst_pad = jnp.full((epad,), n, jnp.int32).at[:e].set(edge_index[1])
    x_pad = jnp.zeros((npad, d), f32).at[:n].set(x)
    y2 = jnp.zeros((npad, 1), jnp.int32).at[:n, 0].set(y)
    zeros_w = jnp.zeros((npad, width), f32)
    zeros_8 = jnp.zeros((npad, 8), f32)
    b2 = b.reshape(1, c)

    tchunk = ntpad // NW // 2
    flagsp = _sc_scatter_add(tidx_pad, m8, zeros_8,
                             npad=npad, width=8, chunk=tchunk)

    h_aug, l_cur, h0_pad = _tc_h1(x_pad, y2, flagsp, W, b2,
                                  d=d, c=c, width=width)

    aggp = None
    for it in range(NUM_ITERS):
        aggp = _sc_gather_scatter_add(h_aug, src_pad, dst_pad, zeros_w,
                                      npad=npad, width=width, chunk=CHUNK)
        if it < NUM_ITERS - 1:
            h_aug, l_cur = _tc_iter(aggp, h0_pad, l_cur, flagsp, W,
                                    d=d, c=c, width=width)

    out_pad = _tc_final(aggp, c=c, width=width)
    return out_pad[:n]


# speedup vs baseline: 5.8804x; 5.8804x over previous
import functools

import jax
import jax.numpy as jnp
from jax import lax
from jax.experimental import pallas as pl
from jax.experimental.pallas import tpu as pltpu
from jax.experimental.pallas import tpu_sc as plsc

SPLIT_RATIO = 0.5
NUM_ITERS = 3

_INFO = plsc.get_sparse_core_info()
NC = _INFO.num_cores
NS = _INFO.num_subcores
NW = NC * NS
CHUNK = 128

ROW_BLK = 1280


def _round_up(v, m):
    return (v + m - 1) // m * m


@functools.partial(jax.jit, static_argnames=("npad", "width", "chunk"))
def _sc_scatter_add(idx_pad, rows_pad, zeros, *, npad, width, chunk):
    epad = idx_pad.shape[0]
    epw = epad // NW
    nchunk = epw // chunk
    zrows = npad // NS

    mesh = plsc.VectorSubcoreMesh(core_axis_name="c", subcore_axis_name="s")

    @functools.partial(
        pl.kernel,
        out_type=jax.ShapeDtypeStruct((NC, npad, width), jnp.float32),
        mesh=mesh,
        scratch_types=[
            pltpu.VMEM((chunk,), jnp.int32),
            pltpu.VMEM((chunk, width), jnp.float32),
            pltpu.VMEM_SHARED((npad, width), jnp.float32),
        ],
        compiler_params=pltpu.CompilerParams(use_tc_tiling_on_sc=False),
    )
    def body(idx_hbm, rows_hbm, zeros_hbm, out_hbm, didx, rows_v, agg_sh):
        cid = lax.axis_index("c")
        sid = lax.axis_index("s")
        wid = sid * NC + cid
        pltpu.sync_copy(zeros_hbm.at[pl.ds(sid * zrows, zrows)],
                        agg_sh.at[pl.ds(sid * zrows, zrows)])
        plsc.subcore_barrier()

        def step(k, carry):
            base = wid * epw + k * chunk
            pltpu.sync_copy(idx_hbm.at[pl.ds(base, chunk)], didx)
            pltpu.sync_copy(rows_hbm.at[pl.ds(base, chunk)], rows_v)
            pltpu.sync_copy(rows_v, agg_sh.at[didx], add=True)
            return carry

        lax.fori_loop(0, nchunk, step, 0)
        plsc.subcore_barrier()
        pltpu.sync_copy(agg_sh.at[pl.ds(sid * zrows, zrows)],
                        out_hbm.at[cid].at[pl.ds(sid * zrows, zrows)])

    return body(idx_pad, rows_pad, zeros)


@functools.partial(jax.jit, static_argnames=("npad", "width", "chunk"))
def _sc_gather_scatter_add(h_aug, src_pad, dst_pad, zeros, *, npad, width, chunk):
    epad = src_pad.shape[0]
    epw = epad // NW
    nchunk = epw // chunk
    zrows = npad // NS

    mesh = plsc.VectorSubcoreMesh(core_axis_name="c", subcore_axis_name="s")

    @functools.partial(
        pl.kernel,
        out_type=jax.ShapeDtypeStruct((NC, npad, width), jnp.float32),
        mesh=mesh,
        scratch_types=[
            pltpu.VMEM((chunk,), jnp.int32),
            pltpu.VMEM((chunk,), jnp.int32),
            pltpu.VMEM((chunk, width), jnp.float32),
            pltpu.VMEM_SHARED((npad, width), jnp.float32),
            pltpu.SemaphoreType.DMA,
        ],
        compiler_params=pltpu.CompilerParams(use_tc_tiling_on_sc=False),
    )
    def body(h_hbm, src_hbm, dst_hbm, zeros_hbm, out_hbm,
             sidx, didx, rows_v, agg_sh, sem):
        cid = lax.axis_index("c")
        sid = lax.axis_index("s")
        wid = sid * NC + cid
        pltpu.sync_copy(zeros_hbm.at[pl.ds(sid * zrows, zrows)],
                        agg_sh.at[pl.ds(sid * zrows, zrows)])
        plsc.subcore_barrier()

        def step(k, carry):
            base = wid * epw + k * chunk
            pltpu.sync_copy(src_hbm.at[pl.ds(base, chunk)], sidx)
            pltpu.sync_copy(dst_hbm.at[pl.ds(base, chunk)], didx)
            pltpu.async_copy(h_hbm.at[sidx], rows_v, sem).wait()
            pltpu.sync_copy(rows_v, agg_sh.at[didx], add=True)
            return carry

        lax.fori_loop(0, nchunk, step, 0)
        plsc.subcore_barrier()
        pltpu.sync_copy(agg_sh.at[pl.ds(sid * zrows, zrows)],
                        out_hbm.at[cid].at[pl.ds(sid * zrows, zrows)])

    return body(h_aug, src_pad, dst_pad, zeros)


def _tc_h1(x_pad, y2, flagsp, W, b2, *, d, c, width):
    npad = x_pad.shape[0]
    grid = npad // ROW_BLK

    def body(x_ref, y_ref, fl_ref, w_ref, b_ref, haug_ref, l_ref, h0_ref):
        xb = x_ref[...]
        w1 = w_ref[:d, :]
        w2 = w_ref[d:, :]
        h0 = jnp.dot(xb, w1, preferred_element_type=jnp.float32) + b_ref[...]
        labeled = (fl_ref[0, :, 0:1] + fl_ref[1, :, 0:1]) > 0.0
        iota = lax.broadcasted_iota(jnp.int32, (ROW_BLK, c), 1)
        onehot = jnp.where(labeled & (y_ref[...] == iota), 1.0, 0.0)
        h = h0 + jnp.dot(onehot, w2, preferred_element_type=jnp.float32)
        haug_ref[...] = jnp.concatenate(
            [h, jnp.ones((ROW_BLK, 1), jnp.float32),
             jnp.zeros((ROW_BLK, width - c - 1), jnp.float32)], axis=1)
        l_ref[...] = onehot
        h0_ref[...] = h0

    return pl.pallas_call(
        body,
        grid=(grid,),
        in_specs=[
            pl.BlockSpec((ROW_BLK, d), lambda i: (i, 0)),
            pl.BlockSpec((ROW_BLK, 1), lambda i: (i, 0)),
            pl.BlockSpec((2, ROW_BLK, 8), lambda i: (0, i, 0)),
            pl.BlockSpec(W.shape, lambda i: (0, 0)),
            pl.BlockSpec(b2.shape, lambda i: (0, 0)),
        ],
        out_specs=[
            pl.BlockSpec((ROW_BLK, width), lambda i: (i, 0)),
            pl.BlockSpec((ROW_BLK, c), lambda i: (i, 0)),
            pl.BlockSpec((ROW_BLK, c), lambda i: (i, 0)),
        ],
        out_shape=[
            jax.ShapeDtypeStruct((npad, width), jnp.float32),
            jax.ShapeDtypeStruct((npad, c), jnp.float32),
            jax.ShapeDtypeStruct((npad, c), jnp.float32),
        ],
    )(x_pad, y2, flagsp, W, b2)


def _tc_iter(aggp, h0_pad, l_pad, flagsp, W, *, d, c, width):
    npad = h0_pad.shape[0]
    grid = npad // ROW_BLK

    def body(agg_ref, h0_ref, l_ref, fl_ref, w_ref, haug_ref, l_out_ref):
        agg = agg_ref[0] + agg_ref[1]
        deg = jnp.maximum(agg[:, c:c + 1], 1.0)
        out = agg[:, :c] / deg
        m = jnp.max(out, axis=1, keepdims=True)
        ex = jnp.exp(out - m)
        p = ex / jnp.sum(ex, axis=1, keepdims=True)
        upd = (fl_ref[0, :, 1:2] + fl_ref[1, :, 1:2]) > 0.0
        l_new = jnp.where(upd, p, l_ref[...])
        w2 = w_ref[d:, :]
        h = h0_ref[...] + jnp.dot(l_new, w2, preferred_element_type=jnp.float32)
        haug_ref[...] = jnp.concatenate(
            [h, jnp.ones((ROW_BLK, 1), jnp.float32),
             jnp.zeros((ROW_BLK, width - c - 1), jnp.float32)], axis=1)
        l_out_ref[...] = l_new

    return pl.pallas_call(
        body,
        grid=(grid,),
        in_specs=[
            pl.BlockSpec((2, ROW_BLK, width), lambda i: (0, i, 0)),
            pl.BlockSpec((ROW_BLK, c), lambda i: (i, 0)),
            pl.BlockSpec((ROW_BLK, c), lambda i: (i, 0)),
            pl.BlockSpec((2, ROW_BLK, 8), lambda i: (0, i, 0)),
            pl.BlockSpec(W.shape, lambda i: (0, 0)),
        ],
        out_specs=[
            pl.BlockSpec((ROW_BLK, width), lambda i: (i, 0)),
            pl.BlockSpec((ROW_BLK, c), lambda i: (i, 0)),
        ],
        out_shape=[
            jax.ShapeDtypeStruct((npad, width), jnp.float32),
            jax.ShapeDtypeStruct((npad, c), jnp.float32),
        ],
    )(aggp, h0_pad, l_pad, flagsp, W)


def _tc_final(aggp, *, c, width):
    npad = aggp.shape[1]
    grid = npad // ROW_BLK

    def body(agg_ref, out_ref):
        agg = agg_ref[0] + agg_ref[1]
        deg = jnp.maximum(agg[:, c:c + 1], 1.0)
        out_ref[...] = agg[:, :c] / deg

    return pl.pallas_call(
        body,
        grid=(grid,),
        in_specs=[pl.BlockSpec((2, ROW_BLK, width), lambda i: (0, i, 0))],
        out_specs=pl.BlockSpec((ROW_BLK, c), lambda i: (i, 0)),
        out_shape=jax.ShapeDtypeStruct((npad, c), jnp.float32),
    )(aggp)


def kernel(x, edge_index, y, train_idx, W, b):
    n, d = x.shape
    c = W.shape[0] - d
    e = edge_index.shape[1]
    nt = train_idx.shape[0]
    width = _round_up(c + 1, 16)
    npad = _round_up(n + 1, ROW_BLK)
    epad = _round_up(e, NW * CHUNK)
    ntpad = _round_up(nt, NW * 16)

    f32 = jnp.float32
    mask = jax.random.uniform(jax.random.key(1), (nt,)) < SPLIT_RATIO
    mask_f = mask.astype(f32)

    tidx_pad = jnp.full((ntpad,), n, jnp.int32).at[:nt].set(train_idx)
    m8 = (jnp.zeros((ntpad, 8), f32)
          .at[:nt, 0].set(mask_f)
          .at[:nt, 1].set(1.0 - mask_f))
    src_pad = jnp.full((epad,), n, jnp.int32).at[:e].set(edge_index[0])
    dst_pad = jnp.full((epad,), n, jnp.int32).at[:e].set(edge_index[1])
    x_pad = jnp.zeros((npad, d), f32).at[:n].set(x)
    y2 = jnp.zeros((npad, 1), jnp.int32).at[:n, 0].set(y)
    zeros_w = jnp.zeros((npad, width), f32)
    zeros_8 = jnp.zeros((npad, 8), f32)
    b2 = b.reshape(1, c)

    tchunk = ntpad // NW // 2
    flagsp = _sc_scatter_add(tidx_pad, m8, zeros_8,
                             npad=npad, width=8, chunk=tchunk)

    h_aug, l_cur, h0_pad = _tc_h1(x_pad, y2, flagsp, W, b2,
                                  d=d, c=c, width=width)

    aggp = None
    for it in range(NUM_ITERS):
        aggp = _sc_gather_scatter_add(h_aug, src_pad, dst_pad, zeros_w,
                                      npad=npad, width=width, chunk=CHUNK)
        if it < NUM_ITERS - 1:
            h_aug, l_cur = _tc_iter(aggp, h0_pad, l_cur, flagsp, W,
                                    d=d, c=c, width=width)

    out_pad = _tc_final(aggp, c=c, width=width)
    return out_pad[:n]


# speedup vs baseline: 6.3250x; 1.0756x over previous
import functools

import jax
import jax.numpy as jnp
from jax import lax
from jax.experimental import pallas as pl
from jax.experimental.pallas import tpu as pltpu
from jax.experimental.pallas import tpu_sc as plsc

SPLIT_RATIO = 0.5
NUM_ITERS = 3

_INFO = plsc.get_sparse_core_info()
NC = _INFO.num_cores
NS = _INFO.num_subcores
NW = NC * NS
CHUNK = 1024

ROW_BLK = 1280


def _round_up(v, m):
    return (v + m - 1) // m * m


@functools.partial(jax.jit, static_argnames=("npad", "width", "chunk"))
def _sc_scatter_add(idx_pad, rows_pad, zeros, *, npad, width, chunk):
    epad = idx_pad.shape[0]
    epw = epad // NW
    nchunk = epw // chunk
    zrows = npad // NS

    mesh = plsc.VectorSubcoreMesh(core_axis_name="c", subcore_axis_name="s")

    @functools.partial(
        pl.kernel,
        out_type=jax.ShapeDtypeStruct((NC, npad, width), jnp.float32),
        mesh=mesh,
        scratch_types=[
            pltpu.VMEM((chunk,), jnp.int32),
            pltpu.VMEM((chunk, width), jnp.float32),
            pltpu.VMEM_SHARED((npad, width), jnp.float32),
        ],
        compiler_params=pltpu.CompilerParams(use_tc_tiling_on_sc=False),
    )
    def body(idx_hbm, rows_hbm, zeros_hbm, out_hbm, didx, rows_v, agg_sh):
        cid = lax.axis_index("c")
        sid = lax.axis_index("s")
        wid = sid * NC + cid
        pltpu.sync_copy(zeros_hbm.at[pl.ds(sid * zrows, zrows)],
                        agg_sh.at[pl.ds(sid * zrows, zrows)])
        plsc.subcore_barrier()

        def step(k, carry):
            base = wid * epw + k * chunk
            pltpu.sync_copy(idx_hbm.at[pl.ds(base, chunk)], didx)
            pltpu.sync_copy(rows_hbm.at[pl.ds(base, chunk)], rows_v)
            pltpu.sync_copy(rows_v, agg_sh.at[didx], add=True)
            return carry

        lax.fori_loop(0, nchunk, step, 0)
        plsc.subcore_barrier()
        pltpu.sync_copy(agg_sh.at[pl.ds(sid * zrows, zrows)],
                        out_hbm.at[cid].at[pl.ds(sid * zrows, zrows)])

    return body(idx_pad, rows_pad, zeros)


GROUP = 1


@functools.partial(jax.jit, static_argnames=("npad", "width", "chunk"))
def _sc_gather_scatter_add(h_aug, src3, dst3, zeros, *, npad, width, chunk):
    nchunk = src3.shape[1]
    epw = nchunk * chunk
    ngroup = nchunk // GROUP
    zrows = npad // NS

    mesh = plsc.VectorSubcoreMesh(core_axis_name="c", subcore_axis_name="s")

    @functools.partial(
        pl.kernel,
        out_type=jax.ShapeDtypeStruct((NC, npad, width), jnp.float32),
        mesh=mesh,
        scratch_types=[
            pltpu.VMEM((chunk,), jnp.int32),
            pltpu.VMEM((chunk,), jnp.int32),
            pltpu.VMEM((chunk, width), jnp.float32),
            pltpu.VMEM_SHARED((npad, width), jnp.float32),
            pltpu.SemaphoreType.DMA,
        ],
        compiler_params=pltpu.CompilerParams(use_tc_tiling_on_sc=False),
    )
    def body(h_hbm, src_hbm, dst_hbm, zeros_hbm, out_hbm,
             sidx, didx, rows_v, agg_sh, sem):
        cid = lax.axis_index("c")
        sid = lax.axis_index("s")
        wid = sid * NC + cid

        pltpu.sync_copy(zeros_hbm.at[pl.ds(sid * zrows, zrows)],
                        agg_sh.at[pl.ds(sid * zrows, zrows)])
        plsc.subcore_barrier()

        def step(g, carry):
            pltpu.sync_copy(src_hbm.at[wid, g], sidx)
            pltpu.sync_copy(dst_hbm.at[wid, g], didx)
            pltpu.async_copy(h_hbm.at[sidx], rows_v, sem).wait()
            pltpu.sync_copy(rows_v, agg_sh.at[didx], add=True)
            return carry

        lax.fori_loop(0, ngroup, step, 0)
        plsc.subcore_barrier()
        pltpu.sync_copy(agg_sh.at[pl.ds(sid * zrows, zrows)],
                        out_hbm.at[cid].at[pl.ds(sid * zrows, zrows)])

    return body(h_aug, src3.reshape(NW, ngroup, chunk),
                dst3.reshape(NW, ngroup, chunk), zeros)


def _tc_h1(x_pad, y2, flagsp, W, b2, *, d, c, width):
    npad = x_pad.shape[0]
    grid = npad // ROW_BLK

    def body(x_ref, y_ref, fl_ref, w_ref, b_ref, haug_ref, l_ref, h0_ref):
        xb = x_ref[...]
        w1 = w_ref[:d, :]
        w2 = w_ref[d:, :]
        h0 = jnp.dot(xb, w1, preferred_element_type=jnp.float32) + b_ref[...]
        labeled = (fl_ref[0, :, 0:1] + fl_ref[1, :, 0:1]) > 0.0
        iota = lax.broadcasted_iota(jnp.int32, (ROW_BLK, c), 1)
        onehot = jnp.where(labeled & (y_ref[...] == iota), 1.0, 0.0)
        h = h0 + jnp.dot(onehot, w2, preferred_element_type=jnp.float32)
        haug_ref[...] = jnp.concatenate(
            [h, jnp.ones((ROW_BLK, 1), jnp.float32),
             jnp.zeros((ROW_BLK, width - c - 1), jnp.float32)], axis=1)
        l_ref[...] = onehot
        h0_ref[...] = h0

    return pl.pallas_call(
        body,
        grid=(grid,),
        in_specs=[
            pl.BlockSpec((ROW_BLK, d), lambda i: (i, 0)),
            pl.BlockSpec((ROW_BLK, 1), lambda i: (i, 0)),
            pl.BlockSpec((2, ROW_BLK, 8), lambda i: (0, i, 0)),
            pl.BlockSpec(W.shape, lambda i: (0, 0)),
            pl.BlockSpec(b2.shape, lambda i: (0, 0)),
        ],
        out_specs=[
            pl.BlockSpec((ROW_BLK, width), lambda i: (i, 0)),
            pl.BlockSpec((ROW_BLK, c), lambda i: (i, 0)),
            pl.BlockSpec((ROW_BLK, c), lambda i: (i, 0)),
        ],
        out_shape=[
            jax.ShapeDtypeStruct((npad, width), jnp.float32),
            jax.ShapeDtypeStruct((npad, c), jnp.float32),
            jax.ShapeDtypeStruct((npad, c), jnp.float32),
        ],
    )(x_pad, y2, flagsp, W, b2)


def _tc_iter(aggp, h0_pad, l_pad, flagsp, W, *, d, c, width):
    npad = h0_pad.shape[0]
    grid = npad // ROW_BLK

    def body(agg_ref, h0_ref, l_ref, fl_ref, w_ref, haug_ref, l_out_ref):
        agg = agg_ref[0] + agg_ref[1]
        deg = jnp.maximum(agg[:, c:c + 1], 1.0)
        out = agg[:, :c] / deg
        m = jnp.max(out, axis=1, keepdims=True)
        ex = jnp.exp(out - m)
        p = ex / jnp.sum(ex, axis=1, keepdims=True)
        upd = (fl_ref[0, :, 1:2] + fl_ref[1, :, 1:2]) > 0.0
        l_new = jnp.where(upd, p, l_ref[...])
        w2 = w_ref[d:, :]
        h = h0_ref[...] + jnp.dot(l_new, w2, preferred_element_type=jnp.float32)
        haug_ref[...] = jnp.concatenate(
            [h, jnp.ones((ROW_BLK, 1), jnp.float32),
             jnp.zeros((ROW_BLK, width - c - 1), jnp.float32)], axis=1)
        l_out_ref[...] = l_new

    return pl.pallas_call(
        body,
        grid=(grid,),
        in_specs=[
            pl.BlockSpec((2, ROW_BLK, width), lambda i: (0, i, 0)),
            pl.BlockSpec((ROW_BLK, c), lambda i: (i, 0)),
            pl.BlockSpec((ROW_BLK, c), lambda i: (i, 0)),
            pl.BlockSpec((2, ROW_BLK, 8), lambda i: (0, i, 0)),
            pl.BlockSpec(W.shape, lambda i: (0, 0)),
        ],
        out_specs=[
            pl.BlockSpec((ROW_BLK, width), lambda i: (i, 0)),
            pl.BlockSpec((ROW_BLK, c), lambda i: (i, 0)),
        ],
        out_shape=[
            jax.ShapeDtypeStruct((npad, width), jnp.float32),
            jax.ShapeDtypeStruct((npad, c), jnp.float32),
        ],
    )(aggp, h0_pad, l_pad, flagsp, W)


def _tc_final(aggp, *, c, width):
    npad = aggp.shape[1]
    grid = npad // ROW_BLK

    def body(agg_ref, out_ref):
        agg = agg_ref[0] + agg_ref[1]
        deg = jnp.maximum(agg[:, c:c + 1], 1.0)
        out_ref[...] = agg[:, :c] / deg

    return pl.pallas_call(
        body,
        grid=(grid,),
        in_specs=[pl.BlockSpec((2, ROW_BLK, width), lambda i: (0, i, 0))],
        out_specs=pl.BlockSpec((ROW_BLK, c), lambda i: (i, 0)),
        out_shape=jax.ShapeDtypeStruct((npad, c), jnp.float32),
    )(aggp)


def kernel(x, edge_index, y, train_idx, W, b):
    n, d = x.shape
    c = W.shape[0] - d
    e = edge_index.shape[1]
    nt = train_idx.shape[0]
    width = _round_up(c + 1, 16)
    npad = _round_up(n + 1, ROW_BLK)
    epad = _round_up(e, NW * CHUNK * GROUP)
    ntpad = _round_up(nt, NW * 16)

    f32 = jnp.float32
    mask = jax.random.uniform(jax.random.key(1), (nt,)) < SPLIT_RATIO
    mask_f = mask.astype(f32)

    tidx_pad = jnp.full((ntpad,), n, jnp.int32).at[:nt].set(train_idx)
    m8 = (jnp.zeros((ntpad, 8), f32)
          .at[:nt, 0].set(mask_f)
          .at[:nt, 1].set(1.0 - mask_f))
    nchunk = epad // NW // CHUNK
    src3 = (jnp.full((epad,), n, jnp.int32).at[:e].set(edge_index[0])
            .reshape(NW, nchunk, CHUNK))
    dst3 = (jnp.full((epad,), n, jnp.int32).at[:e].set(edge_index[1])
            .reshape(NW, nchunk, CHUNK))
    x_pad = jnp.zeros((npad, d), f32).at[:n].set(x)
    y2 = jnp.zeros((npad, 1), jnp.int32).at[:n, 0].set(y)
    zeros_w = jnp.zeros((npad, width), f32)
    zeros_8 = jnp.zeros((npad, 8), f32)
    b2 = b.reshape(1, c)

    tchunk = ntpad // NW // 2
    flagsp = _sc_scatter_add(tidx_pad, m8, zeros_8,
                             npad=npad, width=8, chunk=tchunk)

    h_aug, l_cur, h0_pad = _tc_h1(x_pad, y2, flagsp, W, b2,
                                  d=d, c=c, width=width)

    aggp = None
    for it in range(NUM_ITERS):
        aggp = _sc_gather_scatter_add(h_aug, src3, dst3, zeros_w,
                                      npad=npad, width=width, chunk=CHUNK)
        if it < NUM_ITERS - 1:
            h_aug, l_cur = _tc_iter(aggp, h0_pad, l_cur, flagsp, W,
                                    d=d, c=c, width=width)

    out_pad = _tc_final(aggp, c=c, width=width)
    return out_pad[:n]
